# baseline (speedup 1.0000x reference)
import jax
import jax.numpy as jnp
from jax import lax
from jax.experimental import pallas as pl

_BS = 100
_K = 10


def _gcn_kernel(x_ref, tin_ref, tout_ref, ttr_ref, w_ref, s_ref, out_ref):
    x = x_ref[...]
    alpha = s_ref[0, 0]
    c0 = s_ref[0, 1]
    c1 = s_ref[0, 2]
    c2 = s_ref[0, 3]
    b = s_ref[0, 4]

    xc = x - jnp.mean(x, axis=1, keepdims=True)
    g = lax.dot_general(xc, xc, (((1,), (1,)), ((), ())),
                        preferred_element_type=jnp.float32)
    v = lax.dot_general(x, w_ref[...], (((1,), (1,)), ((), ())),
                        preferred_element_type=jnp.float32)

    inv_col = lax.rsqrt(jnp.sum(xc * xc, axis=1, keepdims=True))
    inv_row = inv_col.T
    ri = lax.broadcasted_iota(jnp.int32, (_BS, _BS), 0)
    ci = lax.broadcasted_iota(jnp.int32, (_BS, _BS), 1)
    eye = jnp.where(ri == ci, jnp.float32(1.0), jnp.float32(0.0))
    pheno = c0 * tin_ref[...] + c1 * tout_ref[...] + c2 * ttr_ref[...] + eye
    pheno_t = pheno.T

    corr = g * inv_col * inv_row
    dist0 = (1.0 - corr) * (1.0 - eye)
    sigma = jnp.mean(dist0)
    inter = jnp.exp(-(dist0 * dist0) / (2.0 * sigma * sigma))
    fea = (inter - eye) * alpha + eye

    adj_t = fea * pheno_t

    neg = jnp.float32(-jnp.inf)
    t = jnp.full((1, _BS), jnp.inf, jnp.float32)
    for _ in range(_K):
        m = jnp.max(jnp.where(adj_t < t, adj_t, neg), axis=0, keepdims=True)
        ge = jnp.sum(jnp.where(adj_t >= t, 1.0, 0.0), axis=0, keepdims=True)
        t = jnp.where(ge <= jnp.float32(_K - 1), m, t)
    adjm_t = jnp.where(adj_t < t, jnp.float32(0.0), adj_t)

    out = jnp.sum(adjm_t * v, axis=0, keepdims=True) + b
    out_ref[...] = out


def kernel(x, alpha, test_in_graph, test_out_graph, train_out_graph, k, c0, c1, c2, W, b):
    del k
    scal = jnp.stack([
        jnp.asarray(alpha, jnp.float32).reshape(()),
        jnp.asarray(c0, jnp.float32).reshape(()),
        jnp.asarray(c1, jnp.float32).reshape(()),
        jnp.asarray(c2, jnp.float32).reshape(()),
        jnp.asarray(b, jnp.float32).reshape(()),
    ]).reshape(1, 5)
    out = pl.pallas_call(
        _gcn_kernel,
        out_shape=jax.ShapeDtypeStruct((1, _BS), jnp.float32),
    )(x, test_in_graph, test_out_graph, train_out_graph, W, scal)
    return out[0]


# speedup vs baseline: 1.1152x; 1.1152x over previous
import jax
import jax.numpy as jnp
from jax import lax
from jax.experimental import pallas as pl

_BS = 100
_K = 10


def _gcn_kernel(x_ref, tin_ref, tout_ref, ttr_ref, w_ref,
                a_ref, c0_ref, c1_ref, c2_ref, b_ref, out_ref):
    x = x_ref[...]
    alpha = a_ref[0, 0].astype(jnp.float32)
    c0 = c0_ref[0, 0]
    c1 = c1_ref[0, 0]
    c2 = c2_ref[0, 0]
    b = b_ref[0, 0]

    xc = x - jnp.mean(x, axis=1, keepdims=True)
    g = lax.dot_general(xc, xc, (((1,), (1,)), ((), ())),
                        preferred_element_type=jnp.float32)
    v = lax.dot_general(x, w_ref[...], (((1,), (1,)), ((), ())),
                        preferred_element_type=jnp.float32)

    inv_col = lax.rsqrt(jnp.sum(xc * xc, axis=1, keepdims=True))
    inv_row = inv_col.T
    ri = lax.broadcasted_iota(jnp.int32, (_BS, _BS), 0)
    ci = lax.broadcasted_iota(jnp.int32, (_BS, _BS), 1)
    eye = jnp.where(ri == ci, jnp.float32(1.0), jnp.float32(0.0))
    pheno = c0 * tin_ref[...] + c1 * tout_ref[...] + c2 * ttr_ref[...] + eye
    pheno_t = pheno.T

    corr = g * inv_col * inv_row
    dist0 = (1.0 - corr) * (1.0 - eye)
    sigma = jnp.mean(dist0)
    inter = jnp.exp(-(dist0 * dist0) / (2.0 * sigma * sigma))
    fea = (inter - eye) * alpha + eye

    adj_t = fea * pheno_t

    neg = jnp.float32(-jnp.inf)
    t = jnp.full((1, _BS), jnp.inf, jnp.float32)
    for _ in range(_K):
        m = jnp.max(jnp.where(adj_t < t, adj_t, neg), axis=0, keepdims=True)
        ge = jnp.sum(jnp.where(adj_t >= t, 1.0, 0.0), axis=0, keepdims=True)
        t = jnp.where(ge <= jnp.float32(_K - 1), m, t)
    adjm_t = jnp.where(adj_t < t, jnp.float32(0.0), adj_t)

    out = jnp.sum(adjm_t * v, axis=0, keepdims=True) + b
    out_ref[...] = out


def kernel(x, alpha, test_in_graph, test_out_graph, train_out_graph, k, c0, c1, c2, W, b):
    del k
    a2 = jnp.reshape(jnp.asarray(alpha), (1, 1))
    out = pl.pallas_call(
        _gcn_kernel,
        out_shape=jax.ShapeDtypeStruct((1, _BS), jnp.float32),
    )(x, test_in_graph, test_out_graph, train_out_graph, W, a2,
      jnp.reshape(c0, (1, 1)), jnp.reshape(c1, (1, 1)),
      jnp.reshape(c2, (1, 1)), jnp.reshape(b, (1, 1)))
    return out[0]
